# baseline (device time: 52647 ns/iter reference)
import jax
import jax.numpy as jnp
from jax import lax
from jax.experimental import pallas as pl
from jax.experimental.pallas import tpu as pltpu

N_DEV = 4


def _gelu(y):
    c = 0.7978845608028654
    return 0.5 * y * (1.0 + jnp.tanh(c * (y + 0.044715 * (y * y * y))))


def kernel(x, w_mat):
    m, _ = x.shape
    _, n = w_mat.shape
    ch = m // N_DEV

    def body(x_ref, w_ref, out_ref, comm_ref, send_sems, recv_sems):
        my = lax.axis_index("i")
        left = lax.rem(my + N_DEV - 1, N_DEV)
        right = lax.rem(my + 1, N_DEV)

        barrier = pltpu.get_barrier_semaphore()
        pl.semaphore_signal(barrier, inc=1, device_id=(left,),
                            device_id_type=pl.DeviceIdType.MESH)
        pl.semaphore_signal(barrier, inc=1, device_id=(right,),
                            device_id_type=pl.DeviceIdType.MESH)
        pl.semaphore_wait(barrier, 2)

        out_ref[...] = jnp.dot(
            x_ref[...].astype(jnp.bfloat16),
            w_ref[...].astype(jnp.bfloat16),
            preferred_element_type=jnp.float32,
        )

        for h in range(N_DEV - 1):
            sc = lax.rem(my + N_DEV - h, N_DEV)
            rc = lax.rem(my + 2 * N_DEV - h - 1, N_DEV)
            comm_ref[6, :, :] = out_ref[pl.ds(sc * ch, ch), :].astype(jnp.bfloat16)
            rdma = pltpu.make_async_remote_copy(
                src_ref=comm_ref.at[6],
                dst_ref=comm_ref.at[h],
                send_sem=send_sems.at[h],
                recv_sem=recv_sems.at[h],
                device_id=(right,),
                device_id_type=pl.DeviceIdType.MESH,
            )
            rdma.start()
            rdma.wait()
            acc = out_ref[pl.ds(rc * ch, ch), :] + comm_ref[h, :, :].astype(jnp.float32)
            out_ref[pl.ds(rc * ch, ch), :] = acc

        oc = lax.rem(my + 1, N_DEV)
        g = _gelu(out_ref[pl.ds(oc * ch, ch), :])
        out_ref[pl.ds(oc * ch, ch), :] = g
        comm_ref[7, :, :] = g.astype(jnp.bfloat16)

        for h in range(N_DEV - 1):
            src_slot = 7 if h == 0 else 3 + h - 1
            rdma = pltpu.make_async_remote_copy(
                src_ref=comm_ref.at[src_slot],
                dst_ref=comm_ref.at[3 + h],
                send_sem=send_sems.at[3 + h],
                recv_sem=recv_sems.at[3 + h],
                device_id=(right,),
                device_id_type=pl.DeviceIdType.MESH,
            )
            rdma.start()
            rdma.wait()
            gc = lax.rem(my + N_DEV - h, N_DEV)
            out_ref[pl.ds(gc * ch, ch), :] = comm_ref[3 + h, :, :].astype(jnp.float32)

    return pl.pallas_call(
        body,
        out_shape=jax.ShapeDtypeStruct((m, n), jnp.float32),
        in_specs=[
            pl.BlockSpec(memory_space=pltpu.VMEM),
            pl.BlockSpec(memory_space=pltpu.VMEM),
        ],
        out_specs=pl.BlockSpec(memory_space=pltpu.VMEM),
        scratch_shapes=[
            pltpu.VMEM((8, ch, n), jnp.bfloat16),
            pltpu.SemaphoreType.DMA((6,)),
            pltpu.SemaphoreType.DMA((6,)),
        ],
        compiler_params=pltpu.CompilerParams(collective_id=0),
    )(x, w_mat)


# device time: 33030 ns/iter; 1.5939x vs baseline; 1.5939x over previous
import jax
import jax.numpy as jnp
from jax import lax
from jax.experimental import pallas as pl
from jax.experimental.pallas import tpu as pltpu

N_DEV = 4


def _gelu(y):
    c = 0.7978845608028654
    return 0.5 * y * (1.0 + jnp.tanh(c * (y + 0.044715 * (y * y * y))))


def kernel(x, w_mat):
    m, _ = x.shape
    _, n = w_mat.shape
    half_r = m // 2
    q_r = m // 4
    half_c = n // 2

    def body(x_ref, w_ref, out_ref,
             sb1, rb1, sb2, rb2, sb3, rb3, sb4, rb4,
             send_sems, recv_sems):
        my = lax.axis_index("i")
        p1 = lax.bitwise_xor(my, 1)
        p2 = 3 - my

        hh = [
            lax.bitwise_and(lax.bitwise_xor(my, lax.shift_right_logical(my, 1)), 1),
            lax.bitwise_and(lax.shift_right_logical(my, 1), 1),
        ]
        qq = [lax.bitwise_and(lax.shift_right_logical(my, 1), 1),
              lax.bitwise_and(my, 1)]
        partners_s1 = [p1, p2]
        partners_s2 = [p2, p1]

        barrier = pltpu.get_barrier_semaphore()
        pl.semaphore_signal(barrier, inc=1, device_id=(p1,),
                            device_id_type=pl.DeviceIdType.MESH)
        pl.semaphore_signal(barrier, inc=1, device_id=(p2,),
                            device_id_type=pl.DeviceIdType.MESH)
        pl.semaphore_wait(barrier, 2)

        def exchange(stage, inst, src, dst, partner):
            rdma = pltpu.make_async_remote_copy(
                src_ref=src, dst_ref=dst,
                send_sem=send_sems.at[stage * 2 + inst],
                recv_sem=recv_sems.at[stage * 2 + inst],
                device_id=(partner,),
                device_id_type=pl.DeviceIdType.MESH,
            )
            rdma.start()
            return rdma

        xb = x_ref[...].astype(jnp.bfloat16)
        wb = w_ref[...].astype(jnp.bfloat16)
        s1 = []
        for inst in range(2):
            cs = inst * half_c
            blk = jnp.dot(xb, wb[:, cs:cs + half_c],
                          preferred_element_type=jnp.float32)
            out_ref[:, cs:cs + half_c] = blk
            send_h = (1 - hh[inst]) * half_r
            sb1[inst, :, :] = out_ref[pl.ds(send_h, half_r),
                                      cs:cs + half_c].astype(jnp.bfloat16)
            s1.append(exchange(0, inst, sb1.at[inst], rb1.at[inst],
                               partners_s1[inst]))
        for inst in range(2):
            s1[inst].wait()

        for inst in range(2):
            cs = inst * half_c
            keep = hh[inst] * half_r
            acc = out_ref[pl.ds(keep, half_r), cs:cs + half_c] \
                + rb1[inst, :, :].astype(jnp.float32)
            out_ref[pl.ds(keep, half_r), cs:cs + half_c] = acc

        s2 = []
        for inst in range(2):
            cs = inst * half_c
            send_q = hh[inst] * half_r + (1 - qq[inst]) * q_r
            sb2[inst, :, :] = out_ref[pl.ds(send_q, q_r),
                                      cs:cs + half_c].astype(jnp.bfloat16)
            s2.append(exchange(1, inst, sb2.at[inst], rb2.at[inst],
                               partners_s2[inst]))
        for inst in range(2):
            s2[inst].wait()

        s3 = []
        for inst in range(2):
            cs = inst * half_c
            keep_q = hh[inst] * half_r + qq[inst] * q_r
            acc = out_ref[pl.ds(keep_q, q_r), cs:cs + half_c] \
                + rb2[inst, :, :].astype(jnp.float32)
            g = _gelu(acc)
            out_ref[pl.ds(keep_q, q_r), cs:cs + half_c] = g
            sb3[inst, :, :] = g.astype(jnp.bfloat16)
            s3.append(exchange(2, inst, sb3.at[inst], rb3.at[inst],
                               partners_s2[inst]))
        for inst in range(2):
            s3[inst].wait()

        s4 = []
        for inst in range(2):
            cs = inst * half_c
            other_q = hh[inst] * half_r + (1 - qq[inst]) * q_r
            out_ref[pl.ds(other_q, q_r), cs:cs + half_c] = \
                rb3[inst, :, :].astype(jnp.float32)
            sb4[inst, :, :] = out_ref[pl.ds(hh[inst] * half_r, half_r),
                                      cs:cs + half_c].astype(jnp.bfloat16)
            s4.append(exchange(3, inst, sb4.at[inst], rb4.at[inst],
                               partners_s1[inst]))
        for inst in range(2):
            s4[inst].wait()
        for inst in range(2):
            cs = inst * half_c
            out_ref[pl.ds((1 - hh[inst]) * half_r, half_r), cs:cs + half_c] = \
                rb4[inst, :, :].astype(jnp.float32)

    return pl.pallas_call(
        body,
        out_shape=jax.ShapeDtypeStruct((m, n), jnp.float32),
        in_specs=[
            pl.BlockSpec(memory_space=pltpu.VMEM),
            pl.BlockSpec(memory_space=pltpu.VMEM),
        ],
        out_specs=pl.BlockSpec(memory_space=pltpu.VMEM),
        scratch_shapes=[
            pltpu.VMEM((2, half_r, half_c), jnp.bfloat16),
            pltpu.VMEM((2, half_r, half_c), jnp.bfloat16),
            pltpu.VMEM((2, q_r, half_c), jnp.bfloat16),
            pltpu.VMEM((2, q_r, half_c), jnp.bfloat16),
            pltpu.VMEM((2, q_r, half_c), jnp.bfloat16),
            pltpu.VMEM((2, q_r, half_c), jnp.bfloat16),
            pltpu.VMEM((2, half_r, half_c), jnp.bfloat16),
            pltpu.VMEM((2, half_r, half_c), jnp.bfloat16),
            pltpu.SemaphoreType.DMA((8,)),
            pltpu.SemaphoreType.DMA((8,)),
        ],
        compiler_params=pltpu.CompilerParams(collective_id=0),
    )(x, w_mat)


# device time: 31726 ns/iter; 1.6594x vs baseline; 1.0411x over previous
import jax
import jax.numpy as jnp
from jax import lax
from jax.experimental import pallas as pl
from jax.experimental.pallas import tpu as pltpu

N_DEV = 4


def _gelu(y):
    c = 0.7978845608028654
    return 0.5 * y * (1.0 + jnp.tanh(c * (y + 0.044715 * (y * y * y))))


def kernel(x, w_mat):
    m, _ = x.shape
    _, n = w_mat.shape
    half_r = m // 2
    q_r = m // 4
    half_c = n // 2

    def body(x_ref, w_ref, out_ref,
             sb1, rb1, sb2, rb2, sb3, rb3, sb4, rb4,
             send_sems, recv_sems):
        my = lax.axis_index("i")
        p1 = lax.bitwise_xor(my, 1)
        p2 = 3 - my

        hh = [
            lax.bitwise_and(lax.bitwise_xor(my, lax.shift_right_logical(my, 1)), 1),
            lax.bitwise_and(lax.shift_right_logical(my, 1), 1),
        ]
        qq = [lax.bitwise_and(lax.shift_right_logical(my, 1), 1),
              lax.bitwise_and(my, 1)]
        partners_s1 = [p1, p2]
        partners_s2 = [p2, p1]

        keep_h = [hh[i] * half_r for i in range(2)]
        send_h = [(1 - hh[i]) * half_r for i in range(2)]
        keep_q = [keep_h[i] + qq[i] * q_r for i in range(2)]
        send_q = [keep_h[i] + (1 - qq[i]) * q_r for i in range(2)]
        cols = [slice(0, half_c), slice(half_c, n)]

        barrier = pltpu.get_barrier_semaphore()
        pl.semaphore_signal(barrier, inc=1, device_id=(p1,),
                            device_id_type=pl.DeviceIdType.MESH)
        pl.semaphore_signal(barrier, inc=1, device_id=(p2,),
                            device_id_type=pl.DeviceIdType.MESH)
        pl.semaphore_wait(barrier, 2)

        def exchange(stage, inst, src, dst, partner):
            rdma = pltpu.make_async_remote_copy(
                src_ref=src, dst_ref=dst,
                send_sem=send_sems.at[stage * 2 + inst],
                recv_sem=recv_sems.at[stage * 2 + inst],
                device_id=(partner,),
                device_id_type=pl.DeviceIdType.MESH,
            )
            rdma.start()
            return rdma

        wb = w_ref[...].astype(jnp.bfloat16)

        s1 = []
        for inst in range(2):
            xb_s = x_ref[pl.ds(send_h[inst], half_r), :].astype(jnp.bfloat16)
            sb1[inst, :, :] = jnp.dot(
                xb_s, wb[:, cols[inst]],
                preferred_element_type=jnp.float32).astype(jnp.bfloat16)
            s1.append(exchange(0, inst, sb1.at[inst], rb1.at[inst],
                               partners_s1[inst]))
        d_qk, d_qs = [], []
        for inst in range(2):
            xb_qk = x_ref[pl.ds(keep_q[inst], q_r), :].astype(jnp.bfloat16)
            xb_qs = x_ref[pl.ds(send_q[inst], q_r), :].astype(jnp.bfloat16)
            d_qk.append(jnp.dot(xb_qk, wb[:, cols[inst]],
                                preferred_element_type=jnp.float32))
            d_qs.append(jnp.dot(xb_qs, wb[:, cols[inst]],
                                preferred_element_type=jnp.float32))

        s2 = []
        for inst in range(2):
            s1[inst].wait()
            p_qs = rb1[inst, pl.ds((1 - qq[inst]) * q_r, q_r), :].astype(jnp.float32)
            sb2[inst, :, :] = (d_qs[inst] + p_qs).astype(jnp.bfloat16)
            s2.append(exchange(1, inst, sb2.at[inst], rb2.at[inst],
                               partners_s2[inst]))
        ksum = []
        for inst in range(2):
            p_qk = rb1[inst, pl.ds(qq[inst] * q_r, q_r), :].astype(jnp.float32)
            ksum.append(d_qk[inst] + p_qk)

        s3 = []
        g = []
        for inst in range(2):
            s2[inst].wait()
            g.append(_gelu(ksum[inst] + rb2[inst, :, :].astype(jnp.float32)))
            sb3[inst, :, :] = g[inst].astype(jnp.bfloat16)
            s3.append(exchange(2, inst, sb3.at[inst], rb3.at[inst],
                               partners_s2[inst]))
        for inst in range(2):
            out_ref[pl.ds(keep_q[inst], q_r), cols[inst]] = g[inst]

        s4 = []
        for inst in range(2):
            s3[inst].wait()
            out_ref[pl.ds(send_q[inst], q_r), cols[inst]] = \
                rb3[inst, :, :].astype(jnp.float32)
            sb4[inst, :, :] = out_ref[pl.ds(keep_h[inst], half_r),
                                      cols[inst]].astype(jnp.bfloat16)
            s4.append(exchange(3, inst, sb4.at[inst], rb4.at[inst],
                               partners_s1[inst]))
        for inst in range(2):
            s4[inst].wait()
            out_ref[pl.ds(send_h[inst], half_r), cols[inst]] = \
                rb4[inst, :, :].astype(jnp.float32)

    return pl.pallas_call(
        body,
        out_shape=jax.ShapeDtypeStruct((m, n), jnp.float32),
        in_specs=[
            pl.BlockSpec(memory_space=pltpu.VMEM),
            pl.BlockSpec(memory_space=pltpu.VMEM),
        ],
        out_specs=pl.BlockSpec(memory_space=pltpu.VMEM),
        scratch_shapes=[
            pltpu.VMEM((2, half_r, half_c), jnp.bfloat16),
            pltpu.VMEM((2, half_r, half_c), jnp.bfloat16),
            pltpu.VMEM((2, q_r, half_c), jnp.bfloat16),
            pltpu.VMEM((2, q_r, half_c), jnp.bfloat16),
            pltpu.VMEM((2, q_r, half_c), jnp.bfloat16),
            pltpu.VMEM((2, q_r, half_c), jnp.bfloat16),
            pltpu.VMEM((2, half_r, half_c), jnp.bfloat16),
            pltpu.VMEM((2, half_r, half_c), jnp.bfloat16),
            pltpu.SemaphoreType.DMA((8,)),
            pltpu.SemaphoreType.DMA((8,)),
        ],
        compiler_params=pltpu.CompilerParams(collective_id=0),
    )(x, w_mat)


# device time: 28616 ns/iter; 1.8398x vs baseline; 1.1087x over previous
import jax
import jax.numpy as jnp
from jax import lax
from jax.experimental import pallas as pl
from jax.experimental.pallas import tpu as pltpu

N_DEV = 4
GRAY = [0, 1, 1, 0]


def _gelu(y):
    c = 0.7978845608028654
    return 0.5 * y * (1.0 + jnp.tanh(c * (y + 0.044715 * (y * y * y))))


def kernel(x, w_mat):
    m, _ = x.shape
    _, n = w_mat.shape
    half_r = m // 2
    q_r = m // 4
    half_c = n // 2

    def body(x_ref, w_ref, out_ref,
             sb1, rb1, sb2, rb2, sb3, rb3, rb4,
             send_sems, recv_sems):
        my = lax.axis_index("i")
        p1t = lax.bitwise_xor(my, 1)
        p2t = 3 - my

        barrier = pltpu.get_barrier_semaphore()
        pl.semaphore_signal(barrier, inc=1, device_id=(p1t,),
                            device_id_type=pl.DeviceIdType.MESH)
        pl.semaphore_signal(barrier, inc=1, device_id=(p2t,),
                            device_id_type=pl.DeviceIdType.MESH)
        pl.semaphore_wait(barrier, 2)

        cols = [slice(0, half_c), slice(half_c, n)]
        wb = [w_ref[:, cols[i]].astype(jnp.bfloat16) for i in range(2)]

        def run(dev):
            p1, p2 = dev ^ 1, 3 - dev
            hh = [GRAY[dev], dev >> 1]
            qq = [dev >> 1, dev & 1]
            s1_partner = [p1, p2]
            s2_partner = [p2, p1]
            KH = [hh[i] * half_r for i in range(2)]
            SH = [(1 - hh[i]) * half_r for i in range(2)]

            def dot_rows(r0, inst):
                xb = x_ref[r0:r0 + q_r, :].astype(jnp.bfloat16)
                return jnp.dot(xb, wb[inst], preferred_element_type=jnp.float32)

            def mk(src, dst, idx, tgt):
                return pltpu.make_async_remote_copy(
                    src_ref=src, dst_ref=dst,
                    send_sem=send_sems.at[idx], recv_sem=recv_sems.at[idx],
                    device_id=(tgt,), device_id_type=pl.DeviceIdType.MESH)

            jf = [1 - qq[0], qq[1]]
            s1d = [[None, None], [None, None]]
            for inst, j in [(0, jf[0]), (1, jf[1]),
                            (0, 1 - jf[0]), (1, 1 - jf[1])]:
                sb1[inst, j] = dot_rows(SH[inst] + j * q_r,
                                        inst).astype(jnp.bfloat16)
                d = mk(sb1.at[inst, j], rb1.at[inst, j],
                       inst * 2 + j, s1_partner[inst])
                d.start()
                s1d[inst][j] = d

            d_qs = [dot_rows(KH[i] + (1 - qq[i]) * q_r, i) for i in range(2)]
            d_qk = [dot_rows(KH[i] + qq[i] * q_r, i) for i in range(2)]

            s2d = []
            for inst in range(2):
                s1d[inst][1 - qq[inst]].wait_recv()
                sb2[inst] = (d_qs[inst] + rb1[inst, 1 - qq[inst]]
                             .astype(jnp.float32)).astype(jnp.bfloat16)
                d = mk(sb2.at[inst], rb2.at[inst], 4 + inst,
                       s2_partner[inst])
                d.start()
                s2d.append(d)
            ksum = []
            for inst in range(2):
                s1d[inst][qq[inst]].wait_recv()
                ksum.append(d_qk[inst] + rb1[inst, qq[inst]]
                            .astype(jnp.float32))

            s3d = []
            s4d = [[None, None], [None, None]]
            gs = []
            for inst in range(2):
                s2d[inst].wait_recv()
                gv = _gelu(ksum[inst] + rb2[inst].astype(jnp.float32))
                gs.append(gv)
                sb3[inst] = gv.astype(jnp.bfloat16)
                d = mk(sb3.at[inst], rb3.at[inst], 6 + inst,
                       s2_partner[inst])
                d.start()
                s3d.append(d)
            for inst in range(2):
                j = qq[inst]
                d = mk(sb3.at[inst], rb4.at[inst, j],
                       8 + inst * 2 + j, s1_partner[inst])
                d.start()
                s4d[inst][j] = d
            for inst in range(2):
                r0 = KH[inst] + qq[inst] * q_r
                out_ref[r0:r0 + q_r, cols[inst]] = gs[inst]

            for inst in range(2):
                s3d[inst].wait_recv()
                j = 1 - qq[inst]
                d = mk(rb3.at[inst], rb4.at[inst, j],
                       8 + inst * 2 + j, s1_partner[inst])
                d.start()
                s4d[inst][j] = d
            for inst in range(2):
                r0 = KH[inst] + (1 - qq[inst]) * q_r
                out_ref[r0:r0 + q_r, cols[inst]] = \
                    rb3[inst].astype(jnp.float32)

            first4 = [qq[0], 1 - qq[1]]
            for inst, j in [(0, first4[0]), (1, first4[1]),
                            (0, 1 - first4[0]), (1, 1 - first4[1])]:
                s4d[inst][j].wait_recv()
                r0 = SH[inst] + j * q_r
                out_ref[r0:r0 + q_r, cols[inst]] = \
                    rb4[inst, j].astype(jnp.float32)

            for inst in range(2):
                for j in range(2):
                    s1d[inst][j].wait_send()
                    s4d[inst][j].wait_send()
                s2d[inst].wait_send()
                s3d[inst].wait_send()

        for dev in range(N_DEV):
            @pl.when(my == dev)
            def _(dev=dev):
                run(dev)

    return pl.pallas_call(
        body,
        out_shape=jax.ShapeDtypeStruct((m, n), jnp.float32),
        in_specs=[
            pl.BlockSpec(memory_space=pltpu.VMEM),
            pl.BlockSpec(memory_space=pltpu.VMEM),
        ],
        out_specs=pl.BlockSpec(memory_space=pltpu.VMEM),
        scratch_shapes=[
            pltpu.VMEM((2, 2, q_r, half_c), jnp.bfloat16),
            pltpu.VMEM((2, 2, q_r, half_c), jnp.bfloat16),
            pltpu.VMEM((2, q_r, half_c), jnp.bfloat16),
            pltpu.VMEM((2, q_r, half_c), jnp.bfloat16),
            pltpu.VMEM((2, q_r, half_c), jnp.bfloat16),
            pltpu.VMEM((2, q_r, half_c), jnp.bfloat16),
            pltpu.VMEM((2, 2, q_r, half_c), jnp.bfloat16),
            pltpu.SemaphoreType.DMA((12,)),
            pltpu.SemaphoreType.DMA((12,)),
        ],
        compiler_params=pltpu.CompilerParams(collective_id=0),
    )(x, w_mat)


# device time: 27386 ns/iter; 1.9224x vs baseline; 1.0449x over previous
import jax
import jax.numpy as jnp
from jax import lax
from jax.experimental import pallas as pl
from jax.experimental.pallas import tpu as pltpu

N_DEV = 4
GRAY = [0, 1, 1, 0]


def _gelu(y):
    c = 0.7978845608028654
    return 0.5 * y * (1.0 + jnp.tanh(c * (y + 0.044715 * (y * y * y))))


def kernel(x, w_mat):
    m, _ = x.shape
    _, n = w_mat.shape
    half_r = m // 2
    q_r = m // 4
    e_r = m // 8
    half_c = n // 2

    def body(x_ref, w_ref, out_ref,
             sb1, rb1, sb2, rb2, sb3, rb3, rb4,
             send_sems, recv_sems):
        my = lax.axis_index("i")
        p1t = lax.bitwise_xor(my, 1)
        p2t = 3 - my

        barrier = pltpu.get_barrier_semaphore()
        pl.semaphore_signal(barrier, inc=1, device_id=(p1t,),
                            device_id_type=pl.DeviceIdType.MESH)
        pl.semaphore_signal(barrier, inc=1, device_id=(p2t,),
                            device_id_type=pl.DeviceIdType.MESH)

        cols = [slice(0, half_c), slice(half_c, n)]
        wb = [w_ref[:, cols[i]].astype(jnp.bfloat16) for i in range(2)]

        def run(dev):
            p1, p2 = dev ^ 1, 3 - dev
            hh = [GRAY[dev], dev >> 1]
            qq = [dev >> 1, dev & 1]
            s1_partner = [p1, p2]
            s2_partner = [p2, p1]
            KH = [hh[i] * half_r for i in range(2)]
            SH = [(1 - hh[i]) * half_r for i in range(2)]

            def dot_rows(r0, nrows, inst):
                xb = x_ref[r0:r0 + nrows, :].astype(jnp.bfloat16)
                return jnp.dot(xb, wb[inst], preferred_element_type=jnp.float32)

            def mk(src, dst, idx, tgt):
                return pltpu.make_async_remote_copy(
                    src_ref=src, dst_ref=dst,
                    send_sem=send_sems.at[idx], recv_sem=recv_sems.at[idx],
                    device_id=(tgt,), device_id_type=pl.DeviceIdType.MESH)

            jf = [1 - qq[0], qq[1]]
            s1d = [[None, None], [None, None]]
            first = True
            for inst, j in [(0, jf[0]), (1, jf[1]),
                            (0, 1 - jf[0]), (1, 1 - jf[1])]:
                sb1[inst, j] = dot_rows(SH[inst] + j * q_r, q_r,
                                        inst).astype(jnp.bfloat16)
                if first:
                    pl.semaphore_wait(barrier, 2)
                    first = False
                d = mk(sb1.at[inst, j], rb1.at[inst, j],
                       inst * 2 + j, s1_partner[inst])
                d.start()
                s1d[inst][j] = d

            d_qs = [dot_rows(KH[i] + (1 - qq[i]) * q_r, q_r, i)
                    for i in range(2)]
            d_qk = [dot_rows(KH[i] + qq[i] * q_r, q_r, i) for i in range(2)]

            s2d = [[None, None], [None, None]]
            for inst in range(2):
                s1d[inst][1 - qq[inst]].wait_recv()
                psum = (d_qs[inst]
                        + rb1[inst, 1 - qq[inst]].astype(jnp.float32))
                for sj in range(2):
                    sb2[inst, sj] = \
                        psum[sj * e_r:(sj + 1) * e_r, :].astype(jnp.bfloat16)
                    d = mk(sb2.at[inst, sj], rb2.at[inst, sj],
                           4 + inst * 2 + sj, s2_partner[inst])
                    d.start()
                    s2d[inst][sj] = d
            ksum = []
            for inst in range(2):
                s1d[inst][qq[inst]].wait_recv()
                ksum.append(d_qk[inst]
                            + rb1[inst, qq[inst]].astype(jnp.float32))

            s3d = [[None, None], [None, None]]
            s4d = [[None] * 4, [None] * 4]
            for sj in range(2):
                for inst in range(2):
                    s2d[inst][sj].wait_recv()
                    gv = _gelu(ksum[inst][sj * e_r:(sj + 1) * e_r, :]
                               + rb2[inst, sj].astype(jnp.float32))
                    sb3[inst, sj] = gv.astype(jnp.bfloat16)
                    d = mk(sb3.at[inst, sj], rb3.at[inst, sj],
                           8 + inst * 2 + sj, s2_partner[inst])
                    d.start()
                    s3d[inst][sj] = d
                    j4 = 2 * qq[inst] + sj
                    d = mk(sb3.at[inst, sj], rb4.at[inst, j4],
                           12 + inst * 4 + j4, s1_partner[inst])
                    d.start()
                    s4d[inst][j4] = d
                    r0 = KH[inst] + qq[inst] * q_r + sj * e_r
                    out_ref[r0:r0 + e_r, cols[inst]] = gv

            for sj in range(2):
                for inst in range(2):
                    s3d[inst][sj].wait_recv()
                    j4 = 2 * (1 - qq[inst]) + sj
                    d = mk(rb3.at[inst, sj], rb4.at[inst, j4],
                           12 + inst * 4 + j4, s1_partner[inst])
                    d.start()
                    s4d[inst][j4] = d
                for inst in range(2):
                    r0 = KH[inst] + (1 - qq[inst]) * q_r + sj * e_r
                    out_ref[r0:r0 + e_r, cols[inst]] = \
                        rb3[inst, sj].astype(jnp.float32)

            qp = [qq[0], 1 - qq[1]]
            arrive = [[2 * qp[i], 2 * qp[i] + 1,
                       2 * (1 - qp[i]), 2 * (1 - qp[i]) + 1]
                      for i in range(2)]
            for k in range(4):
                for inst in range(2):
                    j4 = arrive[inst][k]
                    s4d[inst][j4].wait_recv()
                    r0 = SH[inst] + j4 * e_r
                    out_ref[r0:r0 + e_r, cols[inst]] = \
                        rb4[inst, j4].astype(jnp.float32)

            for inst in range(2):
                for j in range(2):
                    s1d[inst][j].wait_send()
                    s2d[inst][j].wait_send()
                    s3d[inst][j].wait_send()
                for j4 in range(4):
                    s4d[inst][j4].wait_send()

        for dev in range(N_DEV):
            @pl.when(my == dev)
            def _(dev=dev):
                run(dev)

    return pl.pallas_call(
        body,
        out_shape=jax.ShapeDtypeStruct((m, n), jnp.float32),
        in_specs=[
            pl.BlockSpec(memory_space=pltpu.VMEM),
            pl.BlockSpec(memory_space=pltpu.VMEM),
        ],
        out_specs=pl.BlockSpec(memory_space=pltpu.VMEM),
        scratch_shapes=[
            pltpu.VMEM((2, 2, q_r, half_c), jnp.bfloat16),
            pltpu.VMEM((2, 2, q_r, half_c), jnp.bfloat16),
            pltpu.VMEM((2, 2, e_r, half_c), jnp.bfloat16),
            pltpu.VMEM((2, 2, e_r, half_c), jnp.bfloat16),
            pltpu.VMEM((2, 2, e_r, half_c), jnp.bfloat16),
            pltpu.VMEM((2, 2, e_r, half_c), jnp.bfloat16),
            pltpu.VMEM((2, 4, e_r, half_c), jnp.bfloat16),
            pltpu.SemaphoreType.DMA((20,)),
            pltpu.SemaphoreType.DMA((20,)),
        ],
        compiler_params=pltpu.CompilerParams(collective_id=0),
    )(x, w_mat)
